# baseline (speedup 1.0000x reference)
import functools

import jax
import jax.numpy as jnp
from jax import lax
from jax.experimental import pallas as pl
from jax.experimental.pallas import tpu as pltpu
from jax.experimental.pallas import tpu_sc as plsc

B = 16384
L = 16
NW = 32
RPW = B // NW
G = RPW // L
XW = 66
D = 16

_f32 = jnp.float32
_i32 = jnp.int32


def _body(xf_hbm, cfu_hbm, cfi_hbm, nnu_hbm, nni_hbm, nnW_hbm, icW_hbm,
          ucW_hbm, par_hbm, out_hbm,
          x_v, uidx_v, iidx_v, cfu_v, cfi_v, nnu_v, nni_v, out_v,
          par_v, nnW_v, icW_v, ucW_v, sem):
    cid = lax.axis_index("c")
    sid = lax.axis_index("s")
    wid = cid * 16 + sid
    base = wid * RPW

    pltpu.sync_copy(par_hbm, par_v)
    pltpu.sync_copy(nnW_hbm, nnW_v)
    pltpu.sync_copy(icW_hbm, icW_v)
    pltpu.sync_copy(ucW_hbm, ucW_v)
    pltpu.sync_copy(xf_hbm.at[pl.ds(base * XW, RPW * XW)], x_v)

    lanes = lax.iota(_i32, L)

    def build(g, carry):
        flat = (g * L + lanes) * XW
        uf = plsc.load_gather(x_v, [flat])
        vf = plsc.load_gather(x_v, [flat + 1])
        uidx_v[pl.ds(g * L, L)] = uf.astype(_i32)
        iidx_v[pl.ds(g * L, L)] = vf.astype(_i32)
        return carry
    lax.fori_loop(0, G, build, 0)

    c1 = pltpu.async_copy(cfu_hbm.at[uidx_v], cfu_v, sem)
    c2 = pltpu.async_copy(cfi_hbm.at[iidx_v], cfi_v, sem)
    c3 = pltpu.async_copy(nnu_hbm.at[uidx_v], nnu_v, sem)
    c4 = pltpu.async_copy(nni_hbm.at[iidx_v], nni_v, sem)

    pa_nn = par_v[0, :]
    pa_ic = par_v[1, :]
    pa_uc = par_v[2, :]
    pa_ab = par_v[3, :]
    wnnu = jnp.zeros((L,), _f32)
    wnni = jnp.zeros((L,), _f32)
    wic0 = jnp.zeros((L,), _f32)
    wic1 = jnp.zeros((L,), _f32)
    wuc0 = jnp.zeros((L,), _f32)
    wuc1 = jnp.zeros((L,), _f32)
    for k in range(16):
        s_nn = pa_nn[k]
        wnnu = wnnu + s_nn * nnW_v[k, pl.ds(0, L)]
        wnni = wnni + s_nn * nnW_v[k, pl.ds(L, L)]
        s_ic = pa_ic[k]
        wic0 = wic0 + s_ic * icW_v[k, pl.ds(0, L)]
        wic1 = wic1 + s_ic * icW_v[k, pl.ds(L, L)]
        s_uc = pa_uc[k]
        wuc0 = wuc0 + s_uc * ucW_v[k, pl.ds(0, L)]
        wuc1 = wuc1 + s_uc * ucW_v[k, pl.ds(L, L)]
    wfeat = (wic0, wic1, wuc0, wuc1)

    alpha = pa_ab[0]
    bias = (pa_ab[1]
            + jnp.sum(pa_nn * par_v[4, :])
            + jnp.sum(pa_ic * par_v[5, :])
            + jnp.sum(pa_uc * par_v[6, :]))

    def feats(g, carry):
        flat = (g * L + lanes) * XW + 2
        acc = bias + jnp.zeros((L,), _f32)
        for c in range(4):
            for dd in range(16):
                d = c * 16 + dd
                v = plsc.load_gather(x_v, [flat + d])
                acc = acc + wfeat[c][dd] * v
        out_v[pl.ds(g * L, L)] = acc
        return carry
    lax.fori_loop(0, G, feats, 0)

    c1.wait()
    c2.wait()
    c3.wait()
    c4.wait()

    def emb(g, carry):
        acc = out_v[pl.ds(g * L, L)]
        for rr in range(L):
            r = g * L + rr
            cu = cfu_v[r, :]
            ci = cfi_v[r, :]
            nu = nnu_v[r, :]
            ni = nni_v[r, :]
            tot = jnp.sum(alpha * (cu * ci) + wnnu * nu + wnni * ni)
            sel = (lanes == rr).astype(_f32)
            acc = acc + tot * sel
        out_v[pl.ds(g * L, L)] = acc
        return carry
    lax.fori_loop(0, G, emb, 0)

    pltpu.sync_copy(out_v, out_hbm.at[pl.ds(base, RPW)])


_sc_call = functools.partial(
    pl.kernel,
    out_type=jax.ShapeDtypeStruct((B,), _f32),
    mesh=plsc.VectorSubcoreMesh(core_axis_name="c", subcore_axis_name="s",
                                num_cores=2, num_subcores=16),
    compiler_params=pltpu.CompilerParams(needs_layout_passes=False,
                                         use_tc_tiling_on_sc=False),
    scratch_types=[
        pltpu.VMEM((RPW * XW,), _f32),
        pltpu.VMEM((RPW,), _i32),
        pltpu.VMEM((RPW,), _i32),
        pltpu.VMEM((RPW, D), _f32),
        pltpu.VMEM((RPW, D), _f32),
        pltpu.VMEM((RPW, D), _f32),
        pltpu.VMEM((RPW, D), _f32),
        pltpu.VMEM((RPW,), _f32),
        pltpu.VMEM((8, 16), _f32),
        pltpu.VMEM((16, 32), _f32),
        pltpu.VMEM((16, 32), _f32),
        pltpu.VMEM((16, 32), _f32),
        pltpu.SemaphoreType.DMA,
    ],
)(_body)


def kernel(x, cf_user_emb, cf_item_emb, nn_user_emb, nn_item_emb, nn_fc_W,
           nn_fc_b, ic_W, ic_b, uc_W, uc_b, fc_W, fc_b,
           item_context_features_in, user_context_features_in):
    row3 = jnp.concatenate([fc_W[0, 0:1], fc_b, jnp.zeros((14,), _f32)])
    params = jnp.stack([
        fc_W[0, 1:17], fc_W[0, 17:33], fc_W[0, 33:49], row3,
        nn_fc_b, ic_b, uc_b, jnp.zeros((16,), _f32),
    ])
    out = _sc_call(x.reshape(-1), cf_user_emb, cf_item_emb, nn_user_emb,
                   nn_item_emb, nn_fc_W, ic_W, uc_W, params)
    return out[:, None]
